# baseline (speedup 1.0000x reference)
import functools

import jax
import jax.numpy as jnp
from jax import lax
from jax.experimental import pallas as pl
from jax.experimental.pallas import tpu as pltpu
from jax.experimental.pallas import tpu_sc as plsc

_CHUNK = 128


@functools.partial(jax.jit, static_argnums=(2, 3))
def _gather_rows(table, idx2d, B, D):
    info = plsc.get_sparse_core_info()
    NC, NS = info.num_cores, info.num_subcores
    NW = NC * NS
    b_per_w = B // NW
    n_chunks = b_per_w // _CHUNK
    mesh = plsc.VectorSubcoreMesh(core_axis_name="c", subcore_axis_name="s")

    @functools.partial(
        pl.kernel,
        out_type=jax.ShapeDtypeStruct((B, D), jnp.float32),
        mesh=mesh,
        scratch_types=[
            pltpu.VMEM((n_chunks, _CHUNK), jnp.int32),
            pltpu.VMEM((_CHUNK, D), jnp.float32),
            pltpu.SemaphoreType.DMA,
        ],
        compiler_params=pltpu.CompilerParams(use_tc_tiling_on_sc=False),
    )
    def gather_kernel(table_hbm, idx_hbm, out_hbm, idx_v, rows_v, gsem):
        wid = lax.axis_index("s") * NC + lax.axis_index("c")
        base = wid * b_per_w
        pltpu.sync_copy(idx_hbm.at[pl.ds(wid * n_chunks, n_chunks)], idx_v)

        def body(j, carry):
            pltpu.async_copy(table_hbm.at[idx_v.at[j]], rows_v, gsem).wait()
            pltpu.sync_copy(rows_v, out_hbm.at[pl.ds(base + j * _CHUNK, _CHUNK)])
            return carry

        lax.fori_loop(0, n_chunks, body, 0)

    return gather_kernel(table, idx2d)


def kernel(items, table):
    B = items.shape[0] * items.shape[1]
    D = table.shape[1]
    idx2d = items.reshape(B // _CHUNK, _CHUNK).astype(jnp.int32)
    out = _gather_rows(table, idx2d, B, D)
    return out.reshape(items.shape + (D,))


# speedup vs baseline: 1.1170x; 1.1170x over previous
import functools

import jax
import jax.numpy as jnp
from jax import lax
from jax.experimental import pallas as pl
from jax.experimental.pallas import tpu as pltpu
from jax.experimental.pallas import tpu_sc as plsc

_CHUNK = 128
_K = 5


@functools.partial(jax.jit, static_argnums=(2, 3))
def _gather_rows(table, idx2d, B, D):
    info = plsc.get_sparse_core_info()
    NC, NS = info.num_cores, info.num_subcores
    NW = NC * NS
    b_per_w = B // NW
    n_chunks = b_per_w // _CHUNK
    n_groups = n_chunks // _K
    n_pairs = n_groups // 2
    group_rows = _K * _CHUNK
    mesh = plsc.VectorSubcoreMesh(core_axis_name="c", subcore_axis_name="s")

    @functools.partial(
        pl.kernel,
        out_type=jax.ShapeDtypeStruct((B, D), jnp.float32),
        mesh=mesh,
        scratch_types=[
            pltpu.VMEM((n_chunks, _CHUNK), jnp.int32),
            pltpu.VMEM((2, group_rows, D), jnp.float32),
            pltpu.SemaphoreType.DMA,
            pltpu.SemaphoreType.DMA,
            pltpu.SemaphoreType.DMA,
            pltpu.SemaphoreType.DMA,
        ],
        compiler_params=pltpu.CompilerParams(use_tc_tiling_on_sc=False),
    )
    def gather_kernel(table_hbm, idx_hbm, out_hbm, idx_v, rows_v,
                      gsem_a, gsem_b, ssem_a, ssem_b):
        wid = lax.axis_index("s") * NC + lax.axis_index("c")
        base = wid * b_per_w
        pltpu.sync_copy(idx_hbm.at[pl.ds(wid * n_chunks, n_chunks)], idx_v)

        def gather_descs(g, p, sem):
            return [
                pltpu.make_async_copy(
                    table_hbm.at[idx_v.at[g * _K + b]],
                    rows_v.at[p, pl.ds(b * _CHUNK, _CHUNK)],
                    sem,
                )
                for b in range(_K)
            ]

        def store_desc(g, p, sem):
            return pltpu.make_async_copy(
                rows_v.at[p],
                out_hbm.at[pl.ds(base + g * group_rows, group_rows)],
                sem,
            )

        for d in gather_descs(0, 0, gsem_a):
            d.start()

        def body(i, carry):
            g = 2 * i

            @pl.when(i > 0)
            def _():
                store_desc(g - 1, 1, ssem_b).wait()

            for d in gather_descs(g + 1, 1, gsem_b):
                d.start()
            for d in gather_descs(g, 0, gsem_a):
                d.wait()
            store_desc(g, 0, ssem_a).start()
            store_desc(g, 0, ssem_a).wait()

            @pl.when(i + 1 < n_pairs)
            def _():
                for d in gather_descs(g + 2, 0, gsem_a):
                    d.start()

            for d in gather_descs(g + 1, 1, gsem_b):
                d.wait()
            store_desc(g + 1, 1, ssem_b).start()
            return carry

        lax.fori_loop(0, n_pairs, body, 0)
        store_desc(n_groups - 1, 1, ssem_b).wait()

    return gather_kernel(table, idx2d)


def kernel(items, table):
    B = items.shape[0] * items.shape[1]
    D = table.shape[1]
    idx2d = items.reshape(B // _CHUNK, _CHUNK).astype(jnp.int32)
    out = _gather_rows(table, idx2d, B, D)
    return out.reshape(items.shape + (D,))
